# baseline (speedup 1.0000x reference)
import functools

import jax
import jax.numpy as jnp
from jax import lax
from jax.experimental import pallas as pl
from jax.experimental.pallas import tpu as pltpu

_RING = 4
_DELAY = 32
_EPS_CHUNKS = 8


def _round8(x):
    return ((x + 7) // 8) * 8


def _fused_kernel(
    ids_ref,
    pblk_ref,
    packed_hbm,
    eps_hbm,
    kl_ref,
    emb_ref,
    pk_buf,
    eps_vmem,
    etile,
    row_sems,
    eps_sem,
    *, T, tile_v, nv, nt, V, D, eps_rows,
):
    i = pl.program_id(0)

    @pl.when(i == 0)
    def _():
        kl_ref[...] = jnp.zeros_like(kl_ref)
        for k in range(_EPS_CHUNKS):
            lo = k * eps_rows
            rows = min(eps_rows, V - lo)
            if rows > 0:
                pltpu.make_async_copy(
                    eps_hbm.at[pl.ds(lo, rows), :],
                    eps_vmem.at[pl.ds(lo, rows), 0, :],
                    eps_sem).start()

    t_iss = i - _DELAY + (_RING - 1)
    tc = i - _DELAY

    @pl.when(t_iss == 0)
    def _():
        pltpu.make_async_copy(
            eps_hbm, eps_vmem.at[pl.ds(0, V), 0, :], eps_sem).wait()

    @pl.when((t_iss >= 0) & (t_iss < nt))
    def _():
        slot = t_iss % _RING
        base = t_iss * T

        def body(tt, carry):
            for u in range(8):
                t = tt * 8 + u
                r = ids_ref[base + t]
                etile[slot, t, :] = eps_vmem[r, 0]
                pltpu.make_async_copy(
                    packed_hbm.at[pl.ds(r, 1), :],
                    pk_buf.at[slot, pl.ds(t, 1), :],
                    row_sems.at[slot]).start()
            return carry

        lax.fori_loop(0, T // 8, body, 0, unroll=True)

    @pl.when((tc >= 0) & (tc < nt))
    def _():
        slot = tc % _RING
        pltpu.make_async_copy(pk_buf.at[slot], pk_buf.at[slot],
                              row_sems.at[slot]).wait()
        pk = pk_buf[slot].astype(jnp.float32)
        emb = pk[:, :D] + jnp.exp(pk[:, D:]) * etile[slot].astype(jnp.float32)
        emb_ref[...] = emb.astype(emb_ref.dtype)

    @pl.when(i < nv - 1)
    def _():
        blk = pblk_ref[...].astype(jnp.float32)
        mu = blk[:, :D]
        ls = blk[:, D:]
        kl = ls + (0.5 + 0.5 * mu * mu) * jnp.exp(-2.0 * ls)
        part = jnp.sum(kl, axis=0, keepdims=True) - 0.5 * tile_v
        kl_ref[...] = kl_ref[...] + part

    @pl.when(i == nv - 1)
    def _():
        blk = pblk_ref[...].astype(jnp.float32)
        mu = blk[:, :D]
        ls = blk[:, D:]
        kl = ls + (0.5 + 0.5 * mu * mu) * jnp.exp(-2.0 * ls) - 0.5
        rows = (nv - 1) * tile_v + lax.broadcasted_iota(jnp.int32, kl.shape, 0)
        kl = jnp.where(rows < V, kl, 0.0)
        kl_ref[...] = kl_ref[...] + jnp.sum(kl, axis=0, keepdims=True)


def kernel(packed, input_ids, eps):
    V, two_d = packed.shape
    D = two_d // 2
    B, S = input_ids.shape
    N = B * S

    tile_v = 512
    nv = pl.cdiv(V, tile_v)

    T = 256
    nt = pl.cdiv(N, T)
    Np = nt * T
    ids = input_ids.reshape(-1).astype(jnp.int32)
    if Np != N:
        ids = jnp.pad(ids, (0, Np - N))
    ids = jnp.clip(ids, 0, V - 1)
    n_steps = max(nv, nt + _DELAY)

    eps_rows = _round8(pl.cdiv(V, _EPS_CHUNKS))
    Vp = max(eps_rows * _EPS_CHUNKS, V)

    kl_part, emb = pl.pallas_call(
        functools.partial(_fused_kernel, T=T, tile_v=tile_v,
                          nv=nv, nt=nt, V=V, D=D, eps_rows=eps_rows),
        out_shape=[
            jax.ShapeDtypeStruct((1, D), jnp.float32),
            jax.ShapeDtypeStruct((Np, D), packed.dtype),
        ],
        grid_spec=pltpu.PrefetchScalarGridSpec(
            num_scalar_prefetch=1,
            grid=(n_steps,),
            in_specs=[
                pl.BlockSpec((tile_v, two_d),
                             lambda i, ids: (jnp.minimum(i, nv - 1), 0)),
                pl.BlockSpec(memory_space=pl.ANY),
                pl.BlockSpec(memory_space=pl.ANY),
            ],
            out_specs=[
                pl.BlockSpec((1, D), lambda i, ids: (0, 0)),
                pl.BlockSpec((T, D),
                             lambda i, ids: (jnp.clip(i - _DELAY, 0, nt - 1), 0)),
            ],
            scratch_shapes=[
                pltpu.VMEM((_RING, T, two_d), packed.dtype),
                pltpu.VMEM((Vp, 1, D), eps.dtype),
                pltpu.VMEM((_RING, T, D), jnp.float32),
                pltpu.SemaphoreType.DMA((_RING,)),
                pltpu.SemaphoreType.DMA,
            ],
        ),
        compiler_params=pltpu.CompilerParams(
            dimension_semantics=("arbitrary",),
            vmem_limit_bytes=57 * 1024 * 1024,
            disable_bounds_checks=True,
        ),
    )(ids, packed, packed, eps)

    elbo = jnp.sum(kl_part)
    return emb[:N].reshape(B, S, D), elbo


# speedup vs baseline: 1.0463x; 1.0463x over previous
import functools

import jax
import jax.numpy as jnp
from jax import lax
from jax.experimental import pallas as pl
from jax.experimental.pallas import tpu as pltpu

_RING = 4
_DELAY = 32
_EPS_CHUNKS = 8


def _round8(x):
    return ((x + 7) // 8) * 8


def _fused_kernel(
    ids_ref,
    pblk_ref,
    packed_hbm,
    eps_hbm,
    kl_ref,
    emb_ref,
    pk_buf,
    eps_vmem,
    etile,
    row_sems,
    eps_sem,
    *, T, tile_v, nv, nt, V, D, eps_rows,
):
    i = pl.program_id(0)

    @pl.when(i == 0)
    def _():
        kl_ref[...] = jnp.zeros_like(kl_ref)
        for k in range(_EPS_CHUNKS):
            lo = k * eps_rows
            rows = min(eps_rows, V - lo)
            if rows > 0:
                pltpu.make_async_copy(
                    eps_hbm.at[pl.ds(lo, rows), :],
                    eps_vmem.at[pl.ds(lo, rows), 0, :],
                    eps_sem).start()

    t_iss = i - _DELAY + (_RING - 1)
    tc = i - _DELAY

    @pl.when(t_iss == 0)
    def _():
        pltpu.make_async_copy(
            eps_hbm, eps_vmem.at[pl.ds(0, V), 0, :], eps_sem).wait()

    for s in range(_RING):
        @pl.when((t_iss >= 0) & (t_iss < nt) & (t_iss % _RING == s))
        def _(s=s):
            base = t_iss * T

            def body(tt, carry):
                for u in range(8):
                    t = tt * 8 + u
                    r = ids_ref[base + t]
                    etile[s, t, :] = eps_vmem[r, 0]
                    pltpu.make_async_copy(
                        packed_hbm.at[pl.ds(r, 1), :],
                        pk_buf.at[s, pl.ds(t, 1), :],
                        row_sems.at[s]).start()
                return carry

            lax.fori_loop(0, T // 8, body, 0, unroll=True)

    @pl.when((tc >= 0) & (tc < nt))
    def _():
        slot = tc % _RING
        pltpu.make_async_copy(pk_buf.at[slot], pk_buf.at[slot],
                              row_sems.at[slot]).wait()
        pk = pk_buf[slot].astype(jnp.float32)
        emb = pk[:, :D] + jnp.exp(pk[:, D:]) * etile[slot].astype(jnp.float32)
        emb_ref[...] = emb.astype(emb_ref.dtype)

    @pl.when(i < nv - 1)
    def _():
        blk = pblk_ref[...].astype(jnp.float32)
        mu = blk[:, :D]
        ls = blk[:, D:]
        kl = ls + (0.5 + 0.5 * mu * mu) * jnp.exp(-2.0 * ls)
        part = jnp.sum(kl, axis=0, keepdims=True) - 0.5 * tile_v
        kl_ref[...] = kl_ref[...] + part

    @pl.when(i == nv - 1)
    def _():
        blk = pblk_ref[...].astype(jnp.float32)
        mu = blk[:, :D]
        ls = blk[:, D:]
        kl = ls + (0.5 + 0.5 * mu * mu) * jnp.exp(-2.0 * ls) - 0.5
        rows = (nv - 1) * tile_v + lax.broadcasted_iota(jnp.int32, kl.shape, 0)
        kl = jnp.where(rows < V, kl, 0.0)
        kl_ref[...] = kl_ref[...] + jnp.sum(kl, axis=0, keepdims=True)


def kernel(packed, input_ids, eps):
    V, two_d = packed.shape
    D = two_d // 2
    B, S = input_ids.shape
    N = B * S

    tile_v = 512
    nv = pl.cdiv(V, tile_v)

    T = 256
    nt = pl.cdiv(N, T)
    Np = nt * T
    ids = input_ids.reshape(-1).astype(jnp.int32)
    if Np != N:
        ids = jnp.pad(ids, (0, Np - N))
    ids = jnp.clip(ids, 0, V - 1)
    n_steps = max(nv, nt + _DELAY)

    eps_rows = _round8(pl.cdiv(V, _EPS_CHUNKS))
    Vp = max(eps_rows * _EPS_CHUNKS, V)

    kl_part, emb = pl.pallas_call(
        functools.partial(_fused_kernel, T=T, tile_v=tile_v,
                          nv=nv, nt=nt, V=V, D=D, eps_rows=eps_rows),
        out_shape=[
            jax.ShapeDtypeStruct((1, D), jnp.float32),
            jax.ShapeDtypeStruct((Np, D), packed.dtype),
        ],
        grid_spec=pltpu.PrefetchScalarGridSpec(
            num_scalar_prefetch=1,
            grid=(n_steps,),
            in_specs=[
                pl.BlockSpec((tile_v, two_d),
                             lambda i, ids: (jnp.minimum(i, nv - 1), 0)),
                pl.BlockSpec(memory_space=pl.ANY),
                pl.BlockSpec(memory_space=pl.ANY),
            ],
            out_specs=[
                pl.BlockSpec((1, D), lambda i, ids: (0, 0)),
                pl.BlockSpec((T, D),
                             lambda i, ids: (jnp.clip(i - _DELAY, 0, nt - 1), 0)),
            ],
            scratch_shapes=[
                pltpu.VMEM((_RING, T, two_d), packed.dtype),
                pltpu.VMEM((Vp, 1, D), eps.dtype),
                pltpu.VMEM((_RING, T, D), jnp.float32),
                pltpu.SemaphoreType.DMA((_RING,)),
                pltpu.SemaphoreType.DMA,
            ],
        ),
        compiler_params=pltpu.CompilerParams(
            dimension_semantics=("arbitrary",),
            vmem_limit_bytes=57 * 1024 * 1024,
            disable_bounds_checks=True,
        ),
    )(ids, packed, packed, eps)

    elbo = jnp.sum(kl_part)
    return emb[:N].reshape(B, S, D), elbo
